# baseline (speedup 1.0000x reference)
import jax
import jax.numpy as jnp
from jax import lax
from jax.experimental import pallas as pl
from jax.experimental.pallas import tpu as pltpu
from jax.experimental.pallas import tpu_sc as plsc

B, H, W, M = 4, 384, 384, 32
NPIX = H * W
NC, NS, L = 2, 16, 16
TILES_PER_BATCH = NC * NS // B
ROWS_PER_TILE = H // TILES_PER_BATCH
CHUNK = ROWS_PER_TILE * W
VECS_PER_ROW = W // L
MSCALE = 8192.0
NQ = 5 * M


def _body(seg_hbm, mot_hbm, offs_hbm, thing_hbm, stage_hbm,
          seg_v, mot_v, offy_v, offx_v, thing_v,
          acc_p, acc_i, acc_mh, acc_ml,
          tot_i32, red_i32, tbl_a, tbl_b):
    c = lax.axis_index("c")
    s = lax.axis_index("s")
    batch = c * 2 + s // TILES_PER_BATCH
    rb = s % TILES_PER_BATCH
    row0 = rb * ROWS_PER_TILE

    pltpu.sync_copy(seg_hbm.at[batch, 0, pl.ds(row0, ROWS_PER_TILE), :], seg_v)
    pltpu.sync_copy(mot_hbm.at[batch, 0, 0, pl.ds(row0, ROWS_PER_TILE), :], mot_v)

    lane = lax.iota(jnp.int32, L)
    lane_base = lane * M
    zi = jnp.zeros((L,), jnp.int32)
    for k in range(M):
        acc_p[pl.ds(k * L, L)] = zi
        acc_i[pl.ds(k * L, L)] = zi
        acc_mh[pl.ds(k * L, L)] = zi
        acc_ml[pl.ds(k * L, L)] = zi

    @plsc.parallel_loop(0, ROWS_PER_TILE, unroll=2)
    def p1_row(r):
        i_vec = jnp.full((L,), row0 + r, jnp.int32)
        seg_c = seg_v[r, pl.ds(0, L)]
        mot_c = mot_v[r, pl.ds(0, L)]
        for k in range(VECS_PER_ROW):
            if k + 1 < VECS_PER_ROW:
                seg_n = seg_v[r, pl.ds((k + 1) * L, L)]
                mot_n = mot_v[r, pl.ds((k + 1) * L, L)]
            idx = seg_c + lane_base
            t26 = (mot_c * (MSCALE * MSCALE)).astype(jnp.int32)
            mh = t26 >> 13
            ml = t26 & 8191
            plsc.addupdate_scatter(acc_p, [idx], jnp.full((L,), k * 2048 + 1, jnp.int32))
            plsc.addupdate_scatter(acc_i, [idx], i_vec)
            plsc.addupdate_scatter(acc_mh, [idx], mh)
            plsc.addupdate_scatter(acc_ml, [idx], ml)
            if k + 1 < VECS_PER_ROW:
                seg_c, mot_c = seg_n, mot_n

    for k in range(M // L):
        vp = acc_p[pl.ds(k * L, L)]
        vc = vp & 2047
        vj = (vp >> 11) << 4
        vi = acc_i[pl.ds(k * L, L)]
        vh = acc_mh[pl.ds(k * L, L)]
        vl = acc_ml[pl.ds(k * L, L)]
        for l in range(1, L):
            o = l * M + k * L
            vp = acc_p[pl.ds(o, L)]
            c2 = vp & 2047
            vc = vc + c2
            vj = vj + ((vp >> 11) << 4) + l * c2
            vi = vi + acc_i[pl.ds(o, L)]
            vh = vh + acc_mh[pl.ds(o, L)]
            vl = vl + acc_ml[pl.ds(o, L)]
        tot_i32[pl.ds(k * L, L)] = vc
        tot_i32[pl.ds(M + k * L, L)] = vi
        tot_i32[pl.ds(2 * M + k * L, L)] = vj
        tot_i32[pl.ds(3 * M + k * L, L)] = vh
        tot_i32[pl.ds(4 * M + k * L, L)] = vl

    wid = c * NS + s
    pltpu.sync_copy(tot_i32, stage_hbm.at[pl.ds(wid * NQ, NQ)])
    plsc.subcore_barrier()
    w0 = (c * NS + (s // TILES_PER_BATCH) * TILES_PER_BATCH) * NQ
    pltpu.sync_copy(stage_hbm.at[pl.ds(w0, TILES_PER_BATCH * NQ)], red_i32)
    for k in range(5 * M // L):
        v = red_i32[pl.ds(k * L, L)]
        for t in range(1, TILES_PER_BATCH):
            v = v + red_i32[pl.ds(t * NQ + k * L, L)]
        tot_i32[pl.ds(k * L, L)] = v

    for k in range(M // L):
        cntf = tot_i32[pl.ds(k * L, L)].astype(jnp.float32)
        isf = tot_i32[pl.ds(M + k * L, L)].astype(jnp.float32)
        jsf = tot_i32[pl.ds(2 * M + k * L, L)].astype(jnp.float32)
        mhf = tot_i32[pl.ds(3 * M + k * L, L)].astype(jnp.float32)
        mlf = tot_i32[pl.ds(4 * M + k * L, L)].astype(jnp.float32)
        msum = (mhf + mlf * (1.0 / MSCALE)) * (1.0 / MSCALE)
        npx = jnp.maximum(cntf, 1.0)
        mv = jnp.where(msum / npx > 0.5, 1.0, 0.0)
        tbl_a[pl.ds(k * L, L)] = mv * (isf / npx + 1.0)
        tbl_b[pl.ds(k * L, L)] = mv * (jsf / npx)

    lane_f = lane.astype(jnp.float32)
    one = jnp.ones((L,), jnp.float32)
    zero = jnp.zeros((L,), jnp.float32)

    @plsc.parallel_loop(0, ROWS_PER_TILE, unroll=2)
    def p2_row(r):
        i1_f = jnp.full((L,), (row0 + r + 1).astype(jnp.float32), jnp.float32)
        seg1 = seg_v[r, pl.ds(0, L)]
        a0 = plsc.load_gather(tbl_a, [seg1])
        b0 = plsc.load_gather(tbl_b, [seg1])
        seg1 = seg_v[r, pl.ds(L, L)]
        for k in range(VECS_PER_ROW):
            if k + 2 < VECS_PER_ROW:
                seg2 = seg_v[r, pl.ds((k + 2) * L, L)]
            if k + 1 < VECS_PER_ROW:
                a1 = plsc.load_gather(tbl_a, [seg1])
                b1 = plsc.load_gather(tbl_b, [seg1])
            t = jnp.where(a0 > 0.0, one, zero)
            offy_v[r, pl.ds(k * L, L)] = t * (a0 - i1_f)
            offx_v[r, pl.ds(k * L, L)] = b0 - t * (lane_f + float(k * L))
            thing_v[r, pl.ds(k * L, L)] = t
            if k + 1 < VECS_PER_ROW:
                a0, b0 = a1, b1
            if k + 2 < VECS_PER_ROW:
                seg1 = seg2

    pltpu.sync_copy(offy_v, offs_hbm.at[batch, 0, pl.ds(row0, ROWS_PER_TILE), :])
    pltpu.sync_copy(offx_v, offs_hbm.at[batch, 1, pl.ds(row0, ROWS_PER_TILE), :])
    pltpu.sync_copy(thing_v, thing_hbm.at[batch, 0, pl.ds(row0, ROWS_PER_TILE), :])


def kernel(segments, motion):
    mesh = plsc.VectorSubcoreMesh(core_axis_name="c", subcore_axis_name="s",
                                  num_cores=NC, num_subcores=NS)
    offs, thing, _ = pl.kernel(
        _body,
        out_type=(jax.ShapeDtypeStruct((B, 2, H, W), jnp.float32),
                  jax.ShapeDtypeStruct((B, 1, H, W), jnp.float32),
                  jax.ShapeDtypeStruct((NC * NS * NQ,), jnp.int32)),
        mesh=mesh,
        compiler_params=pltpu.CompilerParams(needs_layout_passes=False),
        scratch_types=[
            pltpu.VMEM((ROWS_PER_TILE, W), jnp.int32),
            pltpu.VMEM((ROWS_PER_TILE, W), jnp.float32),
            pltpu.VMEM((ROWS_PER_TILE, W), jnp.float32),
            pltpu.VMEM((ROWS_PER_TILE, W), jnp.float32),
            pltpu.VMEM((ROWS_PER_TILE, W), jnp.float32),
            pltpu.VMEM((M * L,), jnp.int32),
            pltpu.VMEM((M * L,), jnp.int32),
            pltpu.VMEM((M * L,), jnp.int32),
            pltpu.VMEM((M * L,), jnp.int32),
            pltpu.VMEM((NQ,), jnp.int32),
            pltpu.VMEM((TILES_PER_BATCH * NQ,), jnp.int32),
            pltpu.VMEM((M,), jnp.float32),
            pltpu.VMEM((M,), jnp.float32),
        ],
    )(segments, motion)
    return (offs, thing)


# speedup vs baseline: 1.0455x; 1.0455x over previous
import jax
import jax.numpy as jnp
from jax import lax
from jax.experimental import pallas as pl
from jax.experimental.pallas import tpu as pltpu
from jax.experimental.pallas import tpu_sc as plsc

B, H, W, M = 4, 384, 384, 32
NPIX = H * W
NC, NS, L = 2, 16, 16
TILES_PER_BATCH = NC * NS // B
ROWS_PER_TILE = H // TILES_PER_BATCH
CHUNK = ROWS_PER_TILE * W
VECS_PER_ROW = W // L
MSCALE = 8192.0
NQ = 5 * M


def _body(seg_hbm, mot_hbm, offs_hbm, thing_hbm, stage_hbm,
          seg_v, mot_v, offy_v, offx_v, thing_v,
          acc_p, acc_i, acc_mh, acc_ml,
          tot_i32, red_i32, tbl_a, tbl_b, dsem):
    c = lax.axis_index("c")
    s = lax.axis_index("s")
    batch = c * 2 + s // TILES_PER_BATCH
    rb = s % TILES_PER_BATCH
    row0 = rb * ROWS_PER_TILE

    in1 = pltpu.async_copy(
        seg_hbm.at[batch, 0, pl.ds(row0, ROWS_PER_TILE), :], seg_v, dsem)
    in2 = pltpu.async_copy(
        mot_hbm.at[batch, 0, 0, pl.ds(row0, ROWS_PER_TILE), :], mot_v, dsem)

    lane = lax.iota(jnp.int32, L)
    lane_base = lane * M
    zi = jnp.zeros((L,), jnp.int32)
    for k in range(M):
        acc_p[pl.ds(k * L, L)] = zi
        acc_i[pl.ds(k * L, L)] = zi
        acc_mh[pl.ds(k * L, L)] = zi
        acc_ml[pl.ds(k * L, L)] = zi
    in1.wait()
    in2.wait()

    @plsc.parallel_loop(0, ROWS_PER_TILE, unroll=4)
    def p1_row(r):
        i_vec = jnp.full((L,), row0 + r, jnp.int32)
        seg_c = seg_v[r, pl.ds(0, L)]
        mot_c = mot_v[r, pl.ds(0, L)]
        for k in range(VECS_PER_ROW):
            if k + 1 < VECS_PER_ROW:
                seg_n = seg_v[r, pl.ds((k + 1) * L, L)]
                mot_n = mot_v[r, pl.ds((k + 1) * L, L)]
            idx = seg_c + lane_base
            t26 = (mot_c * (MSCALE * MSCALE)).astype(jnp.int32)
            mh = t26 >> 13
            ml = t26 & 8191
            plsc.addupdate_scatter(acc_p, [idx], jnp.full((L,), k * 2048 + 1, jnp.int32))
            plsc.addupdate_scatter(acc_i, [idx], i_vec)
            plsc.addupdate_scatter(acc_mh, [idx], mh)
            plsc.addupdate_scatter(acc_ml, [idx], ml)
            if k + 1 < VECS_PER_ROW:
                seg_c, mot_c = seg_n, mot_n

    for k in range(M // L):
        vp = acc_p[pl.ds(k * L, L)]
        vc = vp & 2047
        vj = (vp >> 11) << 4
        vi = acc_i[pl.ds(k * L, L)]
        vh = acc_mh[pl.ds(k * L, L)]
        vl = acc_ml[pl.ds(k * L, L)]
        for l in range(1, L):
            o = l * M + k * L
            vp = acc_p[pl.ds(o, L)]
            c2 = vp & 2047
            vc = vc + c2
            vj = vj + ((vp >> 11) << 4) + l * c2
            vi = vi + acc_i[pl.ds(o, L)]
            vh = vh + acc_mh[pl.ds(o, L)]
            vl = vl + acc_ml[pl.ds(o, L)]
        tot_i32[pl.ds(k * L, L)] = vc
        tot_i32[pl.ds(M + k * L, L)] = vi
        tot_i32[pl.ds(2 * M + k * L, L)] = vj
        tot_i32[pl.ds(3 * M + k * L, L)] = vh
        tot_i32[pl.ds(4 * M + k * L, L)] = vl

    wid = c * NS + s
    pltpu.sync_copy(tot_i32, stage_hbm.at[pl.ds(wid * NQ, NQ)])
    plsc.subcore_barrier()
    w0 = (c * NS + (s // TILES_PER_BATCH) * TILES_PER_BATCH) * NQ
    pltpu.sync_copy(stage_hbm.at[pl.ds(w0, TILES_PER_BATCH * NQ)], red_i32)
    for k in range(5 * M // L):
        v = red_i32[pl.ds(k * L, L)]
        for t in range(1, TILES_PER_BATCH):
            v = v + red_i32[pl.ds(t * NQ + k * L, L)]
        tot_i32[pl.ds(k * L, L)] = v

    for k in range(M // L):
        cntf = tot_i32[pl.ds(k * L, L)].astype(jnp.float32)
        isf = tot_i32[pl.ds(M + k * L, L)].astype(jnp.float32)
        jsf = tot_i32[pl.ds(2 * M + k * L, L)].astype(jnp.float32)
        mhf = tot_i32[pl.ds(3 * M + k * L, L)].astype(jnp.float32)
        mlf = tot_i32[pl.ds(4 * M + k * L, L)].astype(jnp.float32)
        msum = (mhf + mlf * (1.0 / MSCALE)) * (1.0 / MSCALE)
        npx = jnp.maximum(cntf, 1.0)
        mv = jnp.where(msum / npx > 0.5, 1.0, 0.0)
        tbl_a[pl.ds(k * L, L)] = mv * (isf / npx + 1.0)
        tbl_b[pl.ds(k * L, L)] = mv * (jsf / npx)

    lane_f = lane.astype(jnp.float32)
    one = jnp.ones((L,), jnp.float32)
    zero = jnp.zeros((L,), jnp.float32)

    def p2_row(r):
        i1_f = jnp.full((L,), (row0 + r + 1).astype(jnp.float32), jnp.float32)
        seg1 = seg_v[r, pl.ds(0, L)]
        a0 = plsc.load_gather(tbl_a, [seg1])
        b0 = plsc.load_gather(tbl_b, [seg1])
        seg1 = seg_v[r, pl.ds(L, L)]
        for k in range(VECS_PER_ROW):
            if k + 2 < VECS_PER_ROW:
                seg2 = seg_v[r, pl.ds((k + 2) * L, L)]
            if k + 1 < VECS_PER_ROW:
                a1 = plsc.load_gather(tbl_a, [seg1])
                b1 = plsc.load_gather(tbl_b, [seg1])
            t = jnp.where(a0 > 0.0, one, zero)
            offy_v[r, pl.ds(k * L, L)] = t * (a0 - i1_f)
            offx_v[r, pl.ds(k * L, L)] = b0 - t * (lane_f + float(k * L))
            thing_v[r, pl.ds(k * L, L)] = t
            if k + 1 < VECS_PER_ROW:
                a0, b0 = a1, b1
            if k + 2 < VECS_PER_ROW:
                seg1 = seg2

    HR = ROWS_PER_TILE // 2
    plsc.parallel_loop(0, HR, unroll=4)(p2_row)
    outs = [
        pltpu.async_copy(offy_v.at[pl.ds(0, HR)],
                         offs_hbm.at[batch, 0, pl.ds(row0, HR), :], dsem),
        pltpu.async_copy(offx_v.at[pl.ds(0, HR)],
                         offs_hbm.at[batch, 1, pl.ds(row0, HR), :], dsem),
        pltpu.async_copy(thing_v.at[pl.ds(0, HR)],
                         thing_hbm.at[batch, 0, pl.ds(row0, HR), :], dsem),
    ]
    plsc.parallel_loop(HR, ROWS_PER_TILE, unroll=4)(p2_row)
    outs += [
        pltpu.async_copy(offy_v.at[pl.ds(HR, HR)],
                         offs_hbm.at[batch, 0, pl.ds(row0 + HR, HR), :], dsem),
        pltpu.async_copy(offx_v.at[pl.ds(HR, HR)],
                         offs_hbm.at[batch, 1, pl.ds(row0 + HR, HR), :], dsem),
        pltpu.async_copy(thing_v.at[pl.ds(HR, HR)],
                         thing_hbm.at[batch, 0, pl.ds(row0 + HR, HR), :], dsem),
    ]
    for o in outs:
        o.wait()


def kernel(segments, motion):
    mesh = plsc.VectorSubcoreMesh(core_axis_name="c", subcore_axis_name="s",
                                  num_cores=NC, num_subcores=NS)
    offs, thing, _ = pl.kernel(
        _body,
        out_type=(jax.ShapeDtypeStruct((B, 2, H, W), jnp.float32),
                  jax.ShapeDtypeStruct((B, 1, H, W), jnp.float32),
                  jax.ShapeDtypeStruct((NC * NS * NQ,), jnp.int32)),
        mesh=mesh,
        compiler_params=pltpu.CompilerParams(needs_layout_passes=False),
        scratch_types=[
            pltpu.VMEM((ROWS_PER_TILE, W), jnp.int32),
            pltpu.VMEM((ROWS_PER_TILE, W), jnp.float32),
            pltpu.VMEM((ROWS_PER_TILE, W), jnp.float32),
            pltpu.VMEM((ROWS_PER_TILE, W), jnp.float32),
            pltpu.VMEM((ROWS_PER_TILE, W), jnp.float32),
            pltpu.VMEM((M * L,), jnp.int32),
            pltpu.VMEM((M * L,), jnp.int32),
            pltpu.VMEM((M * L,), jnp.int32),
            pltpu.VMEM((M * L,), jnp.int32),
            pltpu.VMEM((NQ,), jnp.int32),
            pltpu.VMEM((TILES_PER_BATCH * NQ,), jnp.int32),
            pltpu.VMEM((M,), jnp.float32),
            pltpu.VMEM((M,), jnp.float32),
            pltpu.SemaphoreType.DMA,
        ],
    )(segments, motion)
    return (offs, thing)
